# baseline (speedup 1.0000x reference)
import jax
import jax.numpy as jnp
from jax import lax
from jax.experimental import pallas as pl
from jax.experimental.pallas import tpu as pltpu

_B1 = 112


def _head_native_kernel(x_ref, w_ref, b_ref, o_ref):
    tb, T, H = x_ref.shape
    xf = x_ref[...].reshape(tb * T, H)
    z = lax.dot_general(w_ref[...], xf, (((1,), (1,)), ((), ())),
                        preferred_element_type=jnp.float32)
    z = z + b_ref[0]
    p = pl.reciprocal(1.0 + jnp.exp(-z), approx=True)
    o_ref[...] = p.reshape(tb, T).astype(o_ref.dtype)


def _head_dense_kernel(y_ref, w_ref, b_ref, o_ref):
    tb, T = o_ref.shape
    z = jnp.dot(w_ref[...], y_ref[...], preferred_element_type=jnp.float32)
    z = z + b_ref[0]
    p = pl.reciprocal(1.0 + jnp.exp(-z), approx=True)
    o_ref[...] = p.reshape(tb, T).astype(o_ref.dtype)


@jax.jit
def _decision_head_fast(x, weight, bias):
    B, T, H = x.shape
    w = weight.reshape(1, H).astype(x.dtype)
    b1 = bias.reshape((1,)).astype(jnp.float32)
    B2 = B - _B1

    y2 = lax.slice(x, (_B1, 0, 0), (B, T, H)).reshape(B2 * T, H).T

    cp = pltpu.CompilerParams(
        dimension_semantics=("arbitrary",),
        vmem_limit_bytes=56 * 1024 * 1024,
    )

    tb1 = 16
    out1 = pl.pallas_call(
        _head_native_kernel,
        out_shape=jax.ShapeDtypeStruct((_B1, T), x.dtype),
        grid=(pl.cdiv(_B1, tb1),),
        in_specs=[
            pl.BlockSpec((tb1, T, H), lambda i: (i, 0, 0)),
            pl.BlockSpec((1, H), lambda i: (0, 0)),
            pl.BlockSpec(memory_space=pltpu.MemorySpace.SMEM),
        ],
        out_specs=pl.BlockSpec((tb1, T), lambda i: (i, 0)),
        compiler_params=cp,
    )(x, w, b1)

    tb2 = 16
    out2 = pl.pallas_call(
        _head_dense_kernel,
        out_shape=jax.ShapeDtypeStruct((B2, T), x.dtype),
        grid=(pl.cdiv(B2, tb2),),
        in_specs=[
            pl.BlockSpec((H, tb2 * T), lambda i: (0, i)),
            pl.BlockSpec((1, H), lambda i: (0, 0)),
            pl.BlockSpec(memory_space=pltpu.MemorySpace.SMEM),
        ],
        out_specs=pl.BlockSpec((tb2, T), lambda i: (i, 0)),
        compiler_params=cp,
    )(y2, w, b1)

    return jnp.concatenate([out1, out2], axis=0)


def kernel(x, weight, bias):
    return _decision_head_fast(x, weight, bias)


# speedup vs baseline: 1.9500x; 1.9500x over previous
import jax
import jax.numpy as jnp
from jax.experimental import pallas as pl
from jax.experimental.pallas import tpu as pltpu


def _head_kernel(y_ref, w_ref, b_ref, o_ref):
    tb, T = o_ref.shape
    z = jnp.dot(w_ref[...], y_ref[...], preferred_element_type=jnp.float32)
    z = z + b_ref[0]
    p = pl.reciprocal(1.0 + jnp.exp(-z), approx=True)
    o_ref[...] = p.reshape(tb, T).astype(o_ref.dtype)


@jax.jit
def _decision_head_fast(x, weight, bias):
    B, T, H = x.shape
    w = weight.reshape(1, H).astype(x.dtype)
    b1 = bias.reshape((1,)).astype(jnp.float32)

    y = x.reshape(B * T, H).T
    tb = 32
    nB = tb * T
    return pl.pallas_call(
        _head_kernel,
        out_shape=jax.ShapeDtypeStruct((B, T), x.dtype),
        grid=(pl.cdiv(B, tb),),
        in_specs=[
            pl.BlockSpec((H, nB), lambda i: (0, i)),
            pl.BlockSpec((1, H), lambda i: (0, 0)),
            pl.BlockSpec(memory_space=pltpu.MemorySpace.SMEM),
        ],
        out_specs=pl.BlockSpec((tb, T), lambda i: (i, 0)),
        compiler_params=pltpu.CompilerParams(
            dimension_semantics=("arbitrary",),
            vmem_limit_bytes=56 * 1024 * 1024,
        ),
    )(y, w, b1)


def kernel(x, weight, bias):
    return _decision_head_fast(x, weight, bias)


# speedup vs baseline: 1.9671x; 1.0088x over previous
import jax
import jax.numpy as jnp
from jax.experimental import pallas as pl
from jax.experimental.pallas import tpu as pltpu


def _head_kernel(y_ref, w_ref, b_ref, o_ref):
    tb, T = o_ref.shape
    z = jnp.dot(w_ref[...], y_ref[...], preferred_element_type=jnp.float32)
    z = z + b_ref[0]
    p = pl.reciprocal(1.0 + jnp.exp(-z), approx=True)
    o_ref[...] = p.reshape(tb, T).astype(o_ref.dtype)


@jax.jit
def _decision_head_fast(x, weight, bias):
    B, T, H = x.shape
    w = weight.reshape(1, H).astype(x.dtype)
    b1 = bias.reshape((1,)).astype(jnp.float32)

    y = x.reshape(B * T, H).T
    tb = 64
    nB = tb * T
    return pl.pallas_call(
        _head_kernel,
        out_shape=jax.ShapeDtypeStruct((B, T), x.dtype),
        grid=(pl.cdiv(B, tb),),
        in_specs=[
            pl.BlockSpec((H, nB), lambda i: (0, i)),
            pl.BlockSpec((1, H), lambda i: (0, 0)),
            pl.BlockSpec(memory_space=pltpu.MemorySpace.SMEM),
        ],
        out_specs=pl.BlockSpec((tb, T), lambda i: (i, 0)),
        compiler_params=pltpu.CompilerParams(
            dimension_semantics=("arbitrary",),
            vmem_limit_bytes=56 * 1024 * 1024,
        ),
    )(y, w, b1)


def kernel(x, weight, bias):
    return _decision_head_fast(x, weight, bias)
